# Initial kernel scaffold; baseline (speedup 1.0000x reference)
import functools

import jax
import jax.numpy as jnp
from jax import lax
from jax.experimental import pallas as pl
from jax.experimental.pallas import tpu as pltpu
from jax.experimental.pallas import tpu_sc as plsc

N = 10000
E = 320000
NC = 2
NS = 16
NW = NC * NS
EPW = E // NW
CH = 80
NCH = EPW // CH
STRIP = N // NS

_MESH = plsc.VectorSubcoreMesh(core_axis_name="c", subcore_axis_name="s")


def _deg_body(dst3, z, out, dstbuf, ones, acc, sem):
    c = lax.axis_index("c")
    s = lax.axis_index("s")
    wid = s * NC + c
    pltpu.sync_copy(dst3.at[wid], dstbuf)
    @pl.loop(0, CH)
    def _(i):
        ones[i, :] = jnp.ones((16,), jnp.float32)
    pltpu.sync_copy(z.at[pl.ds(s * STRIP, STRIP)], acc.at[pl.ds(s * STRIP, STRIP)])
    plsc.subcore_barrier()

    @pl.loop(0, NCH)
    def _(j):
        pltpu.sync_copy(ones, acc.at[dstbuf.at[j]], add=True)

    plsc.subcore_barrier()
    pltpu.sync_copy(acc.at[pl.ds(s * STRIP, STRIP)],
                    out.at[c, pl.ds(s * STRIP, STRIP)])


_deg = functools.partial(
    pl.kernel,
    out_type=jax.ShapeDtypeStruct((NC, N, 16), jnp.float32),
    mesh=_MESH,
    scratch_types=[
        pltpu.VMEM((NCH, CH), jnp.int32),
        pltpu.VMEM((CH, 16), jnp.float32),
        pltpu.VMEM_SHARED((N, 16), jnp.float32),
        pltpu.SemaphoreType.DMA,
    ],
)(_deg_body)


def _make_scat(D):
    def body(table, src3, dst3, z, out, srcbuf, dstbuf, rows, acc, sem):
        c = lax.axis_index("c")
        s = lax.axis_index("s")
        wid = s * NC + c
        pltpu.sync_copy(src3.at[wid], srcbuf)
        pltpu.sync_copy(dst3.at[wid], dstbuf)
        pltpu.sync_copy(z.at[pl.ds(s * STRIP, STRIP)],
                        acc.at[pl.ds(s * STRIP, STRIP)])
        plsc.subcore_barrier()

        @pl.loop(0, NCH)
        def _(j):
            pltpu.async_copy(table.at[srcbuf.at[j]], rows, sem).wait()
            pltpu.sync_copy(rows, acc.at[dstbuf.at[j]], add=True)

        plsc.subcore_barrier()
        pltpu.sync_copy(acc.at[pl.ds(s * STRIP, STRIP)],
                        out.at[c, pl.ds(s * STRIP, STRIP)])

    return pl.kernel(
        body,
        out_type=jax.ShapeDtypeStruct((NC, N, D), jnp.float32),
        mesh=_MESH,
        scratch_types=[
            pltpu.VMEM((NCH, CH), jnp.int32),
            pltpu.VMEM((NCH, CH), jnp.int32),
            pltpu.VMEM((CH, D), jnp.float32),
            pltpu.VMEM_SHARED((N, D), jnp.float32),
            pltpu.SemaphoreType.DMA,
        ],
    )


_scat128 = _make_scat(128)
_scat64 = _make_scat(64)


_RB = 1000


def _dinv_of(degp_ref):
    deg = degp_ref[0, :, 0:1] + degp_ref[1, :, 0:1] + 1.0
    return lax.rsqrt(deg)


def _mm1_body(degp_ref, x_ref, w_ref, o_ref):
    dinv = _dinv_of(degp_ref)
    o_ref[...] = dinv * jnp.dot(x_ref[...], w_ref[...],
                                preferred_element_type=jnp.float32)


def _mm2_body(degp_ref, p_ref, h1p_ref, w_ref, o_ref):
    dinv = _dinv_of(degp_ref)
    t = p_ref[0] + p_ref[1] + h1p_ref[...]
    h1 = jnp.maximum(dinv * t, 0.0)
    o_ref[...] = dinv * jnp.dot(h1, w_ref[...],
                                preferred_element_type=jnp.float32)


def _emb_body(degp_ref, p_ref, h2p_ref, o_ref):
    dinv = _dinv_of(degp_ref)
    o_ref[...] = dinv * (p_ref[0] + p_ref[1] + h2p_ref[...])


_DRB = 250


def _dec_body(a_ref, bt_ref, o_ref):
    o_ref[...] = jnp.dot(a_ref[...], bt_ref[...],
                         preferred_element_type=jnp.float32)


def _degp_spec():
    return pl.BlockSpec((NC, _RB, 16), lambda i: (0, i, 0))


def kernel(x, edge_index, W1, W2):
    src3 = edge_index[0].reshape(NW, NCH, CH)
    dst3 = edge_index[1].reshape(NW, NCH, CH)
    z16 = jnp.zeros((N, 16), jnp.float32)
    z128 = jnp.zeros((N, 128), jnp.float32)
    z64 = jnp.zeros((N, 64), jnp.float32)

    degp = _deg(dst3, z16)

    grid = N // _RB
    h1p = pl.pallas_call(
        _mm1_body,
        grid=(grid,),
        in_specs=[
            _degp_spec(),
            pl.BlockSpec((_RB, 128), lambda i: (i, 0)),
            pl.BlockSpec((128, 128), lambda i: (0, 0)),
        ],
        out_specs=pl.BlockSpec((_RB, 128), lambda i: (i, 0)),
        out_shape=jax.ShapeDtypeStruct((N, 128), jnp.float32),
    )(degp, x, W1)

    p1 = _scat128(h1p, src3, dst3, z128)

    h2p = pl.pallas_call(
        _mm2_body,
        grid=(grid,),
        in_specs=[
            _degp_spec(),
            pl.BlockSpec((NC, _RB, 128), lambda i: (0, i, 0)),
            pl.BlockSpec((_RB, 128), lambda i: (i, 0)),
            pl.BlockSpec((128, 64), lambda i: (0, 0)),
        ],
        out_specs=pl.BlockSpec((_RB, 64), lambda i: (i, 0)),
        out_shape=jax.ShapeDtypeStruct((N, 64), jnp.float32),
    )(degp, p1, h1p, W2)

    p2 = _scat64(h2p, src3, dst3, z64)

    emb = pl.pallas_call(
        _emb_body,
        grid=(grid,),
        in_specs=[
            _degp_spec(),
            pl.BlockSpec((NC, _RB, 64), lambda i: (0, i, 0)),
            pl.BlockSpec((_RB, 64), lambda i: (i, 0)),
        ],
        out_specs=pl.BlockSpec((_RB, 64), lambda i: (i, 0)),
        out_shape=jax.ShapeDtypeStruct((N, 64), jnp.float32),
    )(degp, p2, h2p)

    embT = emb.T
    recon = pl.pallas_call(
        _dec_body,
        grid=(N // _DRB,),
        in_specs=[
            pl.BlockSpec((_DRB, 64), lambda i: (i, 0)),
            pl.BlockSpec((64, N), lambda i: (0, 0)),
        ],
        out_specs=pl.BlockSpec((_DRB, N), lambda i: (i, 0)),
        out_shape=jax.ShapeDtypeStruct((N, N), jnp.float32),
    )(emb, embT)

    return (recon.reshape(-1), emb)


# speedup vs baseline: 11.0877x; 11.0877x over previous
import functools

import jax
import jax.numpy as jnp
from jax import lax
from jax.experimental import pallas as pl
from jax.experimental.pallas import tpu as pltpu
from jax.experimental.pallas import tpu_sc as plsc

N = 10000
E = 320000
NC = 2
NS = 16
NW = NC * NS
EPW = E // NW
CH = 80
NCH = EPW // CH
NP = 10240
STRIP = NP // NS

_MESH = plsc.VectorSubcoreMesh(core_axis_name="c", subcore_axis_name="s")
_SC_PARAMS = pltpu.CompilerParams(use_tc_tiling_on_sc=False)


def _deg_body(dst3, z, out, dstbuf, ones, acc, sem):
    c = lax.axis_index("c")
    s = lax.axis_index("s")
    wid = s * NC + c
    pltpu.sync_copy(dst3.at[wid], dstbuf)
    @pl.loop(0, CH)
    def _(i):
        ones[i, :] = jnp.ones((16,), jnp.float32)
    pltpu.sync_copy(z.at[pl.ds(s * STRIP, STRIP)], acc.at[pl.ds(s * STRIP, STRIP)])
    plsc.subcore_barrier()

    @pl.loop(0, NCH)
    def _(j):
        pltpu.sync_copy(ones, acc.at[dstbuf.at[j]], add=True)

    plsc.subcore_barrier()
    pltpu.sync_copy(acc.at[pl.ds(s * STRIP, STRIP)],
                    out.at[c, pl.ds(s * STRIP, STRIP)])


_deg = functools.partial(
    pl.kernel,
    out_type=jax.ShapeDtypeStruct((NC, NP, 16), jnp.float32),
    mesh=_MESH,
    scratch_types=[
        pltpu.VMEM((NCH, CH), jnp.int32),
        pltpu.VMEM((CH, 16), jnp.float32),
        pltpu.VMEM_SHARED((NP, 16), jnp.float32),
        pltpu.SemaphoreType.DMA,
    ],
    compiler_params=_SC_PARAMS,
)(_deg_body)


def _make_scat(D):
    def body(table, src3, dst3, z, out, srcbuf, dstbuf, rows, acc, sem):
        c = lax.axis_index("c")
        s = lax.axis_index("s")
        wid = s * NC + c
        pltpu.sync_copy(src3.at[wid], srcbuf)
        pltpu.sync_copy(dst3.at[wid], dstbuf)
        pltpu.sync_copy(z.at[pl.ds(s * STRIP, STRIP)],
                        acc.at[pl.ds(s * STRIP, STRIP)])
        plsc.subcore_barrier()

        @pl.loop(0, NCH)
        def _(j):
            pltpu.async_copy(table.at[srcbuf.at[j]], rows, sem).wait()
            pltpu.sync_copy(rows, acc.at[dstbuf.at[j]], add=True)

        plsc.subcore_barrier()
        pltpu.sync_copy(acc.at[pl.ds(s * STRIP, STRIP)],
                        out.at[c, pl.ds(s * STRIP, STRIP)])

    return pl.kernel(
        body,
        out_type=jax.ShapeDtypeStruct((NC, NP, D), jnp.float32),
        mesh=_MESH,
        scratch_types=[
            pltpu.VMEM((NCH, CH), jnp.int32),
            pltpu.VMEM((NCH, CH), jnp.int32),
            pltpu.VMEM((CH, D), jnp.float32),
            pltpu.VMEM_SHARED((NP, D), jnp.float32),
            pltpu.SemaphoreType.DMA,
        ],
        compiler_params=_SC_PARAMS,
    )


_scat128 = _make_scat(128)
_scat64 = _make_scat(64)


_RB = 1000


def _dinv_of(degp_ref):
    deg = degp_ref[0, :, 0:1] + degp_ref[1, :, 0:1] + 1.0
    return lax.rsqrt(deg)


def _mm1_body(degp_ref, x_ref, w_ref, o_ref):
    dinv = _dinv_of(degp_ref)
    o_ref[...] = dinv * jnp.dot(x_ref[...], w_ref[...],
                                preferred_element_type=jnp.float32)


def _mm2_body(degp_ref, p_ref, h1p_ref, w_ref, o_ref):
    dinv = _dinv_of(degp_ref)
    t = p_ref[0] + p_ref[1] + h1p_ref[...]
    h1 = jnp.maximum(dinv * t, 0.0)
    o_ref[...] = dinv * jnp.dot(h1, w_ref[...],
                                preferred_element_type=jnp.float32)


def _emb_body(degp_ref, p_ref, h2p_ref, o_ref):
    dinv = _dinv_of(degp_ref)
    o_ref[...] = dinv * (p_ref[0] + p_ref[1] + h2p_ref[...])


_DRB = 400


def _dec_body(a_ref, bt_ref, o_ref):
    o_ref[...] = jnp.dot(a_ref[...], bt_ref[...],
                         preferred_element_type=jnp.float32)


def _degp_spec():
    return pl.BlockSpec((NC, _RB, 16), lambda i: (0, i, 0))


def kernel(x, edge_index, W1, W2):
    src3 = edge_index[0].reshape(NW, NCH, CH)
    dst3 = edge_index[1].reshape(NW, NCH, CH)
    z16 = jnp.zeros((NP, 16), jnp.float32)
    z128 = jnp.zeros((NP, 128), jnp.float32)
    z64 = jnp.zeros((NP, 64), jnp.float32)

    degp = _deg(dst3, z16)

    grid = N // _RB
    h1p = pl.pallas_call(
        _mm1_body,
        grid=(grid,),
        in_specs=[
            _degp_spec(),
            pl.BlockSpec((_RB, 128), lambda i: (i, 0)),
            pl.BlockSpec((128, 128), lambda i: (0, 0)),
        ],
        out_specs=pl.BlockSpec((_RB, 128), lambda i: (i, 0)),
        out_shape=jax.ShapeDtypeStruct((N, 128), jnp.float32),
    )(degp, x, W1)

    p1 = _scat128(h1p, src3, dst3, z128)

    h2p = pl.pallas_call(
        _mm2_body,
        grid=(grid,),
        in_specs=[
            _degp_spec(),
            pl.BlockSpec((NC, _RB, 128), lambda i: (0, i, 0)),
            pl.BlockSpec((_RB, 128), lambda i: (i, 0)),
            pl.BlockSpec((128, 64), lambda i: (0, 0)),
        ],
        out_specs=pl.BlockSpec((_RB, 64), lambda i: (i, 0)),
        out_shape=jax.ShapeDtypeStruct((N, 64), jnp.float32),
    )(degp, p1, h1p, W2)

    p2 = _scat64(h2p, src3, dst3, z64)

    emb = pl.pallas_call(
        _emb_body,
        grid=(grid,),
        in_specs=[
            _degp_spec(),
            pl.BlockSpec((NC, _RB, 64), lambda i: (0, i, 0)),
            pl.BlockSpec((_RB, 64), lambda i: (i, 0)),
        ],
        out_specs=pl.BlockSpec((_RB, 64), lambda i: (i, 0)),
        out_shape=jax.ShapeDtypeStruct((N, 64), jnp.float32),
    )(degp, p2, h2p)

    embT = emb.T
    recon = pl.pallas_call(
        _dec_body,
        grid=(N // _DRB,),
        in_specs=[
            pl.BlockSpec((_DRB, 64), lambda i: (i, 0)),
            pl.BlockSpec((64, N), lambda i: (0, 0)),
        ],
        out_specs=pl.BlockSpec((_DRB, N), lambda i: (i, 0)),
        out_shape=jax.ShapeDtypeStruct((N, N), jnp.float32),
    )(emb, embT)

    return (recon.reshape(-1), emb)
